# baseline (speedup 1.0000x reference)
import functools

import jax
import jax.numpy as jnp
from jax import lax
from jax.experimental import pallas as pl
from jax.experimental.pallas import tpu as pltpu
from jax.experimental.pallas import tpu_sc as plsc

_VOCAB = 100000
_HID = 768
_MAXS = 2048
_B = 4
_EPS = 1e-12

_NW = 32
_N = _B * _MAXS
_NP = 4
_SW = _MAXS // _NP
_NPW = _NW // _B
_C = _SW // _NPW

_BN = 512
_SPB = _MAXS // _BN


def _sc_piece_body(piece, full, ids_hbm, tok_hbm, out_hbm,
                   ids_v, buf, gsem, wsem):
    wid = lax.axis_index("s") * 2 + lax.axis_index("c")
    b = wid // _NPW
    s = piece * _SW + lax.rem(wid, _NPW) * _C
    pltpu.sync_copy(ids_hbm.at[b, pl.ds(s, _C)], ids_v)
    pltpu.async_copy(tok_hbm.at[ids_v], buf, gsem).wait()
    obase = b * _MAXS + s if full else wid * _C
    pltpu.async_copy(buf, out_hbm.at[pl.ds(obase, _C)], wsem).wait()


def _tc_ln_body(emb_ref, pos_ref, sid_ref, seg_ref, gam_ref, bet_ref, out_ref):
    e = emb_ref[...] + pos_ref[...]
    sidf = sid_ref[0].astype(jnp.float32)
    s0 = seg_ref[0:1, :]
    s1 = seg_ref[1:2, :]
    e = e + s0 + sidf.reshape(_BN, 1) * (s1 - s0)
    mean = jnp.mean(e, axis=-1, keepdims=True)
    var = jnp.mean((e - mean) ** 2, axis=-1, keepdims=True)
    normed = (e - mean) * lax.rsqrt(var + _EPS)
    out_ref[...] = normed * gam_ref[...].reshape(1, _HID) + bet_ref[...].reshape(1, _HID)


def _tc_ln_chain_body(big_ref, emb_ref, pos_ref, sid_ref, seg_ref, gam_ref,
                      bet_ref, out_ref):
    del big_ref
    _tc_ln_body(emb_ref, pos_ref, sid_ref, seg_ref, gam_ref, bet_ref, out_ref)


def _common_specs(piece):
    return [
        pl.BlockSpec((_SW, _HID), lambda g: (piece, 0)),
        pl.BlockSpec((1, 1, _BN), lambda g: (_NP * g + piece, 0, 0)),
        pl.BlockSpec((2, _HID), lambda g: (0, 0)),
        pl.BlockSpec((_HID,), lambda g: (0,)),
        pl.BlockSpec((_HID,), lambda g: (0,)),
    ]


@jax.jit
def _run(input_ids, sid3, token_table, pos_table, seg_table, gamma, beta):
    mesh = plsc.VectorSubcoreMesh(core_axis_name="c", subcore_axis_name="s")
    scratch = [
        pltpu.VMEM((_C,), jnp.int32),
        pltpu.VMEM((_C, _HID), jnp.float32),
        pltpu.SemaphoreType.DMA,
        pltpu.SemaphoreType.DMA,
    ]
    cp = pltpu.CompilerParams(needs_layout_passes=False)

    def gather_piece(piece, full):
        shape = (_N if full else _B * _SW, _HID)
        return pl.kernel(
            functools.partial(_sc_piece_body, piece, full),
            out_type=jax.ShapeDtypeStruct(shape, jnp.float32),
            mesh=mesh, compiler_params=cp, scratch_types=scratch,
        )(input_ids, token_table)

    big = gather_piece(0, True)
    rows = [gather_piece(i, False) for i in range(1, _NP)]

    big = pl.pallas_call(
        _tc_ln_body,
        out_shape=jax.ShapeDtypeStruct((_N, _HID), jnp.float32),
        grid=(_B,),
        in_specs=[pl.BlockSpec((_BN, _HID), lambda g: (_SPB * g, 0))]
        + _common_specs(0),
        out_specs=pl.BlockSpec((_BN, _HID), lambda g: (_SPB * g, 0)),
        input_output_aliases={0: 0},
    )(big, pos_table, sid3, seg_table, gamma, beta)

    for i in range(1, _NP):
        big = pl.pallas_call(
            _tc_ln_chain_body,
            out_shape=jax.ShapeDtypeStruct((_N, _HID), jnp.float32),
            grid=(_B,),
            in_specs=[pl.BlockSpec((8, 128), lambda g: (0, 0)),
                      pl.BlockSpec((_BN, _HID), lambda g: (g, 0))]
            + _common_specs(i),
            out_specs=pl.BlockSpec(
                (_BN, _HID), functools.partial(lambda i_, g: (_SPB * g + i_, 0), i)),
            input_output_aliases={0: 0},
        )(big, rows[i - 1], pos_table, sid3, seg_table, gamma, beta)
    return big


def kernel(input_ids, segment_ids, token_table, pos_table, seg_table, gamma, beta):
    sid3 = segment_ids.reshape(_N // _BN, 1, _BN)
    out = _run(input_ids, sid3, token_table, pos_table, seg_table, gamma, beta)
    return out.reshape(_B, _MAXS, _HID)


# speedup vs baseline: 1.1075x; 1.1075x over previous
import functools

import jax
import jax.numpy as jnp
from jax import lax
from jax.experimental import pallas as pl
from jax.experimental.pallas import tpu as pltpu
from jax.experimental.pallas import tpu_sc as plsc

_VOCAB = 100000
_HID = 768
_MAXS = 2048
_B = 4
_EPS = 1e-12

_NW = 32
_N = _B * _MAXS
_NP = 2
_SW = _MAXS // _NP
_NPW = _NW // _B
_TPW = _SW // _NPW
_C = 64
_NCH = _TPW // _C

_BN = 1024


def _sc_piece_body(piece, full, ids_hbm, tok_hbm, out_hbm,
                   ids0, ids1, buf0, buf1, gsem0, gsem1, wsem0, wsem1):
    wid = lax.axis_index("s") * 2 + lax.axis_index("c")
    b = wid // _NPW
    s = piece * _SW + lax.rem(wid, _NPW) * _TPW
    obase = b * _MAXS + s if full else wid * _TPW
    idbufs = (ids0, ids1)
    bufs = (buf0, buf1)
    gsems = (gsem0, gsem1)
    wsems = (wsem0, wsem1)

    pltpu.sync_copy(ids_hbm.at[b, pl.ds(s, _C)], ids0)
    gathers = [pltpu.async_copy(tok_hbm.at[ids0], buf0, gsem0), None]
    writes = [None, None]
    for g in range(_NCH):
        p = g % 2
        np_ = (g + 1) % 2
        if g + 1 < _NCH:
            if writes[np_] is not None:
                writes[np_].wait()
                writes[np_] = None
            pltpu.sync_copy(ids_hbm.at[b, pl.ds(s + (g + 1) * _C, _C)],
                            idbufs[np_])
            gathers[np_] = pltpu.async_copy(
                tok_hbm.at[idbufs[np_]], bufs[np_], gsems[np_])
        gathers[p].wait()
        writes[p] = pltpu.async_copy(
            bufs[p], out_hbm.at[pl.ds(obase + g * _C, _C)], wsems[p])
    for p in range(2):
        if writes[p] is not None:
            writes[p].wait()


def _ln_block(emb, pos_ref, sid_ref, seg_ref, gam_ref, bet_ref):
    e = emb + pos_ref[...]
    sidf = sid_ref[0].astype(jnp.float32)
    s0 = seg_ref[0:1, :]
    s1 = seg_ref[1:2, :]
    e = e + s0 + sidf.reshape(_BN, 1) * (s1 - s0)
    mean = jnp.mean(e, axis=-1, keepdims=True)
    var = jnp.mean((e - mean) ** 2, axis=-1, keepdims=True)
    normed = (e - mean) * lax.rsqrt(var + _EPS)
    return normed * gam_ref[...].reshape(1, _HID) + bet_ref[...].reshape(1, _HID)


def _tc_ln0_body(emb_ref, pos_ref, sid_ref, seg_ref, gam_ref, bet_ref, out_ref):
    out_ref[...] = _ln_block(emb_ref[...], pos_ref, sid_ref, seg_ref,
                             gam_ref, bet_ref)


def _tc_ln1_body(big_ref, emb_ref, pos_ref, sid_ref, seg_ref, gam_ref, bet_ref,
                 out_ref):
    del big_ref
    out_ref[...] = _ln_block(emb_ref[...], pos_ref, sid_ref, seg_ref,
                             gam_ref, bet_ref)


def _common_specs(piece):
    return [
        pl.BlockSpec((_SW, _HID), lambda g: (piece, 0)),
        pl.BlockSpec((1, 1, _BN), lambda g: (_NP * g + piece, 0, 0)),
        pl.BlockSpec((2, _HID), lambda g: (0, 0)),
        pl.BlockSpec((_HID,), lambda g: (0,)),
        pl.BlockSpec((_HID,), lambda g: (0,)),
    ]


@jax.jit
def _run(input_ids, sid3, token_table, pos_table, seg_table, gamma, beta):
    mesh = plsc.VectorSubcoreMesh(core_axis_name="c", subcore_axis_name="s")
    scratch = [
        pltpu.VMEM((_C,), jnp.int32),
        pltpu.VMEM((_C,), jnp.int32),
        pltpu.VMEM((_C, _HID), jnp.float32),
        pltpu.VMEM((_C, _HID), jnp.float32),
        pltpu.SemaphoreType.DMA,
        pltpu.SemaphoreType.DMA,
        pltpu.SemaphoreType.DMA,
        pltpu.SemaphoreType.DMA,
    ]
    cp = pltpu.CompilerParams(needs_layout_passes=False)

    def gather_piece(piece, full):
        shape = (_N if full else _B * _SW, _HID)
        return pl.kernel(
            functools.partial(_sc_piece_body, piece, full),
            out_type=jax.ShapeDtypeStruct(shape, jnp.float32),
            mesh=mesh, compiler_params=cp, scratch_types=scratch,
        )(input_ids, token_table)

    big = gather_piece(0, True)
    rows1 = gather_piece(1, False)

    big = pl.pallas_call(
        _tc_ln0_body,
        out_shape=jax.ShapeDtypeStruct((_N, _HID), jnp.float32),
        grid=(_B,),
        in_specs=[pl.BlockSpec((_BN, _HID), lambda g: (_NP * g, 0))]
        + _common_specs(0),
        out_specs=pl.BlockSpec((_BN, _HID), lambda g: (_NP * g, 0)),
        input_output_aliases={0: 0},
    )(big, pos_table, sid3, seg_table, gamma, beta)

    out = pl.pallas_call(
        _tc_ln1_body,
        out_shape=jax.ShapeDtypeStruct((_N, _HID), jnp.float32),
        grid=(_B,),
        in_specs=[pl.BlockSpec((8, 128), lambda g: (0, 0)),
                  pl.BlockSpec((_BN, _HID), lambda g: (g, 0))]
        + _common_specs(1),
        out_specs=pl.BlockSpec((_BN, _HID), lambda g: (_NP * g + 1, 0)),
        input_output_aliases={0: 0},
    )(big, rows1, pos_table, sid3, seg_table, gamma, beta)
    return out


def kernel(input_ids, segment_ids, token_table, pos_table, seg_table, gamma, beta):
    sid3 = segment_ids.reshape(_N // _BN, 1, _BN)
    out = _run(input_ids, sid3, token_table, pos_table, seg_table, gamma, beta)
    return out.reshape(_B, _MAXS, _HID)
